# Initial kernel scaffold; baseline (speedup 1.0000x reference)
import jax
import jax.numpy as jnp
from jax.experimental import pallas as pl


def kernel(x, edge_index, W1, att_src1, att_dst1, b1, W2, att_src2, att_dst2, b2):
    raise NotImplementedError("write your pallas kernel here")


# speedup vs baseline: 45.1740x; 45.1740x over previous
import functools

import jax
import jax.numpy as jnp
from jax import lax
from jax.experimental import pallas as pl
from jax.experimental.pallas import tpu as pltpu
from jax.experimental.pallas import tpu_sc as plsc

N = 10000
E = 320000
IN_CH = 128
HID = 16
HEADS = 8
OUT_CH = 64

_HIGH = jax.lax.Precision.HIGHEST
_BM = 1000
_G = N // _BM
CH = 128
NCH = E // CH
NFULL = N // CH
NTAIL = N - NFULL * CH
NT = NCH // 32 + 1

def _sc_mesh():
    return plsc.VectorSubcoreMesh(core_axis_name="c", subcore_axis_name="s")


def _dot(a, b):
    return jnp.dot(a, b, precision=_HIGH, preferred_element_type=jnp.float32)


def _tc1_body(x_ref, w_ref, as_ref, ad_ref, h_ref, q_ref, r_ref, qm_ref, rm_ref):
    i = pl.program_id(0)
    h = _dot(x_ref[...], w_ref[...])
    h_ref[...] = h
    q = _dot(h, as_ref[...])
    r = _dot(h, ad_ref[...])
    q_ref[...] = q
    r_ref[...] = r
    qm = jnp.max(q, axis=0, keepdims=True)
    rm = jnp.max(r, axis=0, keepdims=True)

    @pl.when(i == 0)
    def _():
        qm_ref[...] = qm
        rm_ref[...] = rm

    @pl.when(i > 0)
    def _():
        qm_ref[...] = jnp.maximum(qm_ref[...], qm)
        rm_ref[...] = jnp.maximum(rm_ref[...], rm)


def _tc2_body(acc_ref, den_ref, b1_ref, exp_ref, w2_ref, as_ref, ad_ref,
              h2_ref, s_ref, d_ref, sm_ref, dm_ref):
    i = pl.program_id(0)
    acc = acc_ref[0] + acc_ref[1]
    den8 = den_ref[0][:, :8] + den_ref[1][:, :8]
    den128 = _dot(den8, exp_ref[...])
    h1 = acc / (den128 + 1e-16) + b1_ref[...]
    h2 = _dot(h1, w2_ref[...])
    h2_ref[...] = h2
    s = _dot(h2, as_ref[...])
    d = _dot(h2, ad_ref[...])
    s_ref[...] = s
    d_ref[...] = d
    sm = jnp.max(s, axis=0, keepdims=True)
    dm = jnp.max(d, axis=0, keepdims=True)

    @pl.when(i == 0)
    def _():
        sm_ref[...] = sm
        dm_ref[...] = dm

    @pl.when(i > 0)
    def _():
        sm_ref[...] = jnp.maximum(sm_ref[...], sm)
        dm_ref[...] = jnp.maximum(dm_ref[...], dm)


def _tc3_body(acc_ref, den_ref, b2_ref, out_ref):
    acc = acc_ref[0] + acc_ref[1]
    den = den_ref[0][:, 0:1] + den_ref[1][:, 0:1]
    o = acc / (den + 1e-16) + b2_ref[...]
    m = jnp.max(o, axis=1, keepdims=True)
    z = o - m
    lse = jnp.log(jnp.sum(jnp.exp(z), axis=1, keepdims=True))
    out_ref[...] = z - lse


def _tc1(x, W1, As1, Ad1):
    return pl.pallas_call(
        _tc1_body,
        grid=(_G,),
        in_specs=[
            pl.BlockSpec((_BM, IN_CH), lambda i: (i, 0)),
            pl.BlockSpec((IN_CH, IN_CH), lambda i: (0, 0)),
            pl.BlockSpec((IN_CH, 16), lambda i: (0, 0)),
            pl.BlockSpec((IN_CH, 16), lambda i: (0, 0)),
        ],
        out_specs=[
            pl.BlockSpec((_BM, IN_CH), lambda i: (i, 0)),
            pl.BlockSpec((_BM, 16), lambda i: (i, 0)),
            pl.BlockSpec((_BM, 16), lambda i: (i, 0)),
            pl.BlockSpec((1, 16), lambda i: (0, 0)),
            pl.BlockSpec((1, 16), lambda i: (0, 0)),
        ],
        out_shape=[
            jax.ShapeDtypeStruct((N, IN_CH), jnp.float32),
            jax.ShapeDtypeStruct((N, 16), jnp.float32),
            jax.ShapeDtypeStruct((N, 16), jnp.float32),
            jax.ShapeDtypeStruct((1, 16), jnp.float32),
            jax.ShapeDtypeStruct((1, 16), jnp.float32),
        ],
    )(x, W1, As1, Ad1)


def _tc2(acc1, den1, b1, Exp8, W2, As2, Ad2):
    return pl.pallas_call(
        _tc2_body,
        grid=(_G,),
        in_specs=[
            pl.BlockSpec((2, _BM, IN_CH), lambda i: (0, i, 0)),
            pl.BlockSpec((2, _BM, 16), lambda i: (0, i, 0)),
            pl.BlockSpec((1, IN_CH), lambda i: (0, 0)),
            pl.BlockSpec((8, IN_CH), lambda i: (0, 0)),
            pl.BlockSpec((IN_CH, OUT_CH), lambda i: (0, 0)),
            pl.BlockSpec((OUT_CH, 16), lambda i: (0, 0)),
            pl.BlockSpec((OUT_CH, 16), lambda i: (0, 0)),
        ],
        out_specs=[
            pl.BlockSpec((_BM, OUT_CH), lambda i: (i, 0)),
            pl.BlockSpec((_BM, 16), lambda i: (i, 0)),
            pl.BlockSpec((_BM, 16), lambda i: (i, 0)),
            pl.BlockSpec((1, 16), lambda i: (0, 0)),
            pl.BlockSpec((1, 16), lambda i: (0, 0)),
        ],
        out_shape=[
            jax.ShapeDtypeStruct((N, OUT_CH), jnp.float32),
            jax.ShapeDtypeStruct((N, 16), jnp.float32),
            jax.ShapeDtypeStruct((N, 16), jnp.float32),
            jax.ShapeDtypeStruct((1, 16), jnp.float32),
            jax.ShapeDtypeStruct((1, 16), jnp.float32),
        ],
    )(acc1, den1, b1, Exp8, W2, As2, Ad2)


def _tc3(acc2, den2, b2):
    return pl.pallas_call(
        _tc3_body,
        grid=(_G,),
        in_specs=[
            pl.BlockSpec((2, _BM, OUT_CH), lambda i: (0, i, 0)),
            pl.BlockSpec((2, _BM, 16), lambda i: (0, i, 0)),
            pl.BlockSpec((1, OUT_CH), lambda i: (0, 0)),
        ],
        out_specs=pl.BlockSpec((_BM, OUT_CH), lambda i: (i, 0)),
        out_shape=jax.ShapeDtypeStruct((N, OUT_CH), jnp.float32),
    )(acc2, den2, b2)


def _sc_edge_pass(h, Q, R, C, src, dst, F):
    ngrp = F // 16

    @functools.partial(
        pl.kernel,
        out_type=[
            jax.ShapeDtypeStruct((2, N, F), jnp.float32),
            jax.ShapeDtypeStruct((2, N, 16), jnp.float32),
        ],
        mesh=_sc_mesh(),
        compiler_params=pltpu.CompilerParams(use_tc_tiling_on_sc=False),
        scratch_types=[
            pltpu.VMEM((CH,), jnp.int32),
            pltpu.VMEM((CH,), jnp.int32),
            pltpu.VMEM((CH, 16), jnp.float32),
            pltpu.VMEM((CH, 16), jnp.float32),
            pltpu.VMEM((CH, F), jnp.float32),
            pltpu.VMEM((CH, 16), jnp.float32),
            pltpu.VMEM((1, 16), jnp.float32),
            pltpu.VMEM_SHARED((N, F), jnp.float32),
            pltpu.VMEM_SHARED((N, 16), jnp.float32),
            pltpu.SemaphoreType.DMA,
        ],
    )
    def k(h_hbm, q_hbm, r_hbm, c_hbm, s_hbm, d_hbm, acc_out, den_out,
          sidx, didx, qs, rd, hs, exb, cvec, acc_sp, den_sp, sem):
        cid = lax.axis_index("c")
        sid = lax.axis_index("s")
        wid = sid * 2 + cid

        @pl.loop(0, CH)
        def _(r2):
            for j in range(ngrp):
                hs[r2, pl.ds(j * 16, 16)] = jnp.zeros((16,), jnp.float32)
            exb[r2, :] = jnp.zeros((16,), jnp.float32)

        for j in range(5):
            i = j * 16 + sid

            @pl.when(i < NFULL)
            def _():
                pltpu.sync_copy(hs, acc_sp.at[pl.ds(i * CH, CH)])
                pltpu.sync_copy(exb, den_sp.at[pl.ds(i * CH, CH)])

        @pl.when(sid == 15)
        def _():
            pltpu.sync_copy(hs.at[pl.ds(0, NTAIL)],
                            acc_sp.at[pl.ds(NFULL * CH, NTAIL)])
            pltpu.sync_copy(exb.at[pl.ds(0, NTAIL)],
                            den_sp.at[pl.ds(NFULL * CH, NTAIL)])

        plsc.subcore_barrier()

        pltpu.sync_copy(c_hbm, cvec)
        cv = cvec[0, :]

        @pl.loop(0, NT)
        def _(it):
            t = wid + it * 32

            @pl.when(t < NCH)
            def _():
                off = t * CH
                pltpu.sync_copy(s_hbm.at[pl.ds(off, CH)], sidx)
                pltpu.sync_copy(d_hbm.at[pl.ds(off, CH)], didx)
                g1 = pltpu.async_copy(q_hbm.at[sidx], qs, sem)
                g2 = pltpu.async_copy(r_hbm.at[didx], rd, sem)
                g3 = pltpu.async_copy(h_hbm.at[sidx], hs, sem)
                g1.wait()
                g2.wait()
                g3.wait()

                @pl.loop(0, CH)
                def _(e):
                    a = qs[e, :] + rd[e, :]
                    al = jnp.maximum(a, 0.2 * a)
                    exv = jnp.exp(al - cv)
                    exb[e, :] = exv
                    for g in range(ngrp):
                        head = g if F == IN_CH else 0
                        sp = jnp.full((16,), exv[head], jnp.float32)
                        hs[e, pl.ds(g * 16, 16)] = hs[e, pl.ds(g * 16, 16)] * sp

                pltpu.sync_copy(exb, den_sp.at[didx], add=True)
                pltpu.sync_copy(hs, acc_sp.at[didx], add=True)

        plsc.subcore_barrier()

        for j in range(5):
            i = j * 16 + sid

            @pl.when(i < NFULL)
            def _():
                pltpu.sync_copy(acc_sp.at[pl.ds(i * CH, CH)],
                                acc_out.at[cid, pl.ds(i * CH, CH)])
                pltpu.sync_copy(den_sp.at[pl.ds(i * CH, CH)],
                                den_out.at[cid, pl.ds(i * CH, CH)])

        @pl.when(sid == 15)
        def _():
            pltpu.sync_copy(acc_sp.at[pl.ds(NFULL * CH, NTAIL)],
                            acc_out.at[cid, pl.ds(NFULL * CH, NTAIL)])
            pltpu.sync_copy(den_sp.at[pl.ds(NFULL * CH, NTAIL)],
                            den_out.at[cid, pl.ds(NFULL * CH, NTAIL)])

    return k(h, Q, R, C, src, dst)


def _lrelu(x):
    return jnp.maximum(x, 0.2 * x)


def kernel(x, edge_index, W1, att_src1, att_dst1, b1, W2, att_src2, att_dst2, b2):
    src = edge_index[0].astype(jnp.int32)
    dst = edge_index[1].astype(jnp.int32)

    lane = jnp.arange(16, dtype=jnp.int32) % 8
    grp = jnp.arange(IN_CH, dtype=jnp.int32) // HID
    onehot1 = (grp[:, None] == lane[None, :]).astype(jnp.float32)
    As1 = onehot1 * att_src1.reshape(IN_CH)[:, None]
    Ad1 = onehot1 * att_dst1.reshape(IN_CH)[:, None]
    As2 = jnp.broadcast_to(att_src2.reshape(OUT_CH)[:, None], (OUT_CH, 16))
    Ad2 = jnp.broadcast_to(att_dst2.reshape(OUT_CH)[:, None], (OUT_CH, 16))
    Exp8 = (jnp.arange(8, dtype=jnp.int32)[:, None]
            == grp[None, :]).astype(jnp.float32)

    h1, Q1, R1, QM1, RM1 = _tc1(x, W1, As1, Ad1)
    C1 = _lrelu(QM1 + RM1)
    acc1, den1 = _sc_edge_pass(h1, Q1, R1, C1, src, dst, IN_CH)

    h2, S2, D2, SM2, DM2 = _tc2(acc1, den1, b1.reshape(1, IN_CH), Exp8,
                                W2, As2, Ad2)
    C2 = _lrelu(SM2 + DM2)
    acc2, den2 = _sc_edge_pass(h2, S2, D2, C2, src, dst, OUT_CH)

    return _tc3(acc2, den2, b2.reshape(1, OUT_CH))
